# baseline (device time: 18713 ns/iter reference)
import jax
import jax.numpy as jnp
from jax import lax
from jax.experimental import pallas as pl
from jax.experimental.pallas import tpu as pltpu

N_DEV = 32


def kernel(A, B):
    m, _ = A.shape
    _, n = B.shape
    m_out = m // N_DEV

    def body(a_ref, b_ref, out_ref, chunks_ref, recv_ref,
             send_sems, recv_sems, ready_sems):
        my = lax.axis_index("i")

        barrier_sem = pltpu.get_barrier_semaphore()
        pl.semaphore_signal(barrier_sem, inc=1)
        pl.semaphore_wait(barrier_sem, 1)

        for k in range(1, N_DEV):
            p = lax.rem(my + k, N_DEV)
            pl.semaphore_signal(
                ready_sems.at[my], inc=1,
                device_id=(p,), device_id_type=pl.DeviceIdType.MESH,
            )

        partial = jnp.dot(
            a_ref[...].astype(jnp.bfloat16),
            b_ref[...].astype(jnp.bfloat16),
            preferred_element_type=jnp.float32,
        )
        chunks_ref[...] = partial.reshape(N_DEV, m_out, n).astype(jnp.bfloat16)

        recv_ref[pl.ds(my, 1)] = chunks_ref[pl.ds(my, 1)]

        sends = []
        for k in range(1, N_DEV):
            p = lax.rem(my + k, N_DEV)
            pl.semaphore_wait(ready_sems.at[p], 1)
            rdma = pltpu.make_async_remote_copy(
                src_ref=chunks_ref.at[p],
                dst_ref=recv_ref.at[my],
                send_sem=send_sems.at[k],
                recv_sem=recv_sems.at[my],
                device_id=(p,),
                device_id_type=pl.DeviceIdType.MESH,
            )
            rdma.start()
            sends.append(rdma)

        for k in range(1, N_DEV):
            q = lax.rem(my + k, N_DEV)
            recv = pltpu.make_async_remote_copy(
                src_ref=chunks_ref.at[q],
                dst_ref=recv_ref.at[q],
                send_sem=send_sems.at[0],
                recv_sem=recv_sems.at[q],
                device_id=(q,),
                device_id_type=pl.DeviceIdType.MESH,
            )
            recv.wait_recv()

        out_ref[...] = jnp.sum(recv_ref[...].astype(jnp.float32), axis=0)

        for rdma in sends:
            rdma.wait_send()

    return pl.pallas_call(
        body,
        out_shape=jax.ShapeDtypeStruct((m_out, n), jnp.float32),
        in_specs=[
            pl.BlockSpec(memory_space=pltpu.VMEM),
            pl.BlockSpec(memory_space=pltpu.VMEM),
        ],
        out_specs=pl.BlockSpec(memory_space=pltpu.VMEM),
        scratch_shapes=[
            pltpu.VMEM((N_DEV, m_out, n), jnp.bfloat16),
            pltpu.VMEM((N_DEV, m_out, n), jnp.bfloat16),
            pltpu.SemaphoreType.DMA((N_DEV,)),
            pltpu.SemaphoreType.DMA((N_DEV,)),
            pltpu.SemaphoreType.REGULAR((N_DEV,)),
        ],
        compiler_params=pltpu.CompilerParams(collective_id=0),
    )(A, B)


# device time: 17781 ns/iter; 1.0524x vs baseline; 1.0524x over previous
import jax
import jax.numpy as jnp
from jax import lax
from jax.experimental import pallas as pl
from jax.experimental.pallas import tpu as pltpu

N_DEV = 32
G_SZ = 8
N_GRP = N_DEV // G_SZ
D_PER = G_SZ // N_GRP
N_COLL = N_DEV // G_SZ


def kernel(A, B):
    m, _ = A.shape
    _, n = B.shape
    m_out = m // N_DEV

    def body(
        a_ref, b_ref, out_ref,
        chunks_ref, s1_recv_ref, gchunks_ref, s2_recv_ref,
        s1_send_sems, s1_recv_sems, s2_send_sems, s2_recv_sems,
    ):
        my = lax.axis_index("i")
        slot = lax.rem(my, G_SZ)
        grp = lax.div(my, G_SZ)
        base = my - slot
        coll = lax.div(my, N_COLL)

        barrier_sem = pltpu.get_barrier_semaphore()
        for k in range(1, G_SZ):
            pl.semaphore_signal(
                barrier_sem, inc=1,
                device_id=(base + lax.rem(slot + k, G_SZ),),
                device_id_type=pl.DeviceIdType.MESH,
            )
        for k in range(1, N_GRP):
            pl.semaphore_signal(
                barrier_sem, inc=1,
                device_id=(lax.rem(grp + k, N_GRP) * G_SZ + coll,),
                device_id_type=pl.DeviceIdType.MESH,
            )

        partial = jnp.dot(
            a_ref[...].astype(jnp.bfloat16),
            b_ref[...].astype(jnp.bfloat16),
            preferred_element_type=jnp.float32,
        )
        chunks_ref[...] = partial.reshape(
            G_SZ, N_COLL, m_out, n
        ).astype(jnp.bfloat16)

        s1_recv_ref[pl.ds(slot, 1)] = chunks_ref[pl.ds(slot, 1)]

        targets_local = lax.div(slot, 2) == grp

        @pl.when(targets_local)
        def _():
            pl.semaphore_wait(barrier_sem, G_SZ - 1)

        @pl.when(jnp.logical_not(targets_local))
        def _():
            pl.semaphore_wait(barrier_sem, G_SZ - 1 + N_COLL)

        sends = []
        for k in range(1, G_SZ):
            j = lax.rem(slot + k, G_SZ)
            rdma = pltpu.make_async_remote_copy(
                src_ref=chunks_ref.at[j],
                dst_ref=s1_recv_ref.at[slot],
                send_sem=s1_send_sems.at[k],
                recv_sem=s1_recv_sems.at[slot],
                device_id=(base + j,),
                device_id_type=pl.DeviceIdType.MESH,
            )
            rdma.start()
            sends.append(rdma)

        for k in range(1, G_SZ):
            j = lax.rem(slot + k, G_SZ)
            recv = pltpu.make_async_remote_copy(
                src_ref=chunks_ref.at[j],
                dst_ref=s1_recv_ref.at[j],
                send_sem=s1_send_sems.at[0],
                recv_sem=s1_recv_sems.at[j],
                device_id=(base + j,),
                device_id_type=pl.DeviceIdType.MESH,
            )
            recv.wait_recv()

        gsum = jnp.sum(s1_recv_ref[...].astype(jnp.float32), axis=0)
        gchunks_ref[...] = gsum.astype(jnp.bfloat16)

        for r in range(N_COLL):
            d = N_COLL * slot + r

            @pl.when(d != my)
            def _():
                rdma = pltpu.make_async_remote_copy(
                    src_ref=gchunks_ref.at[r],
                    dst_ref=s2_recv_ref.at[grp],
                    send_sem=s2_send_sems.at[r],
                    recv_sem=s2_recv_sems.at[grp],
                    device_id=(d,),
                    device_id_type=pl.DeviceIdType.MESH,
                )
                rdma.start()

            @pl.when(d == my)
            def _():
                s2_recv_ref[pl.ds(grp, 1)] = gchunks_ref[pl.ds(r, 1)]

        for g in range(N_GRP):
            x = g * G_SZ + coll

            @pl.when(x != my)
            def _():
                recv = pltpu.make_async_remote_copy(
                    src_ref=gchunks_ref.at[0],
                    dst_ref=s2_recv_ref.at[g],
                    send_sem=s2_send_sems.at[0],
                    recv_sem=s2_recv_sems.at[g],
                    device_id=(x,),
                    device_id_type=pl.DeviceIdType.MESH,
                )
                recv.wait_recv()

        out_ref[...] = jnp.sum(s2_recv_ref[...].astype(jnp.float32), axis=0)

        for rdma in sends:
            rdma.wait_send()
        for r in range(N_COLL):
            d = N_COLL * slot + r

            @pl.when(d != my)
            def _():
                wait = pltpu.make_async_remote_copy(
                    src_ref=gchunks_ref.at[r],
                    dst_ref=s2_recv_ref.at[grp],
                    send_sem=s2_send_sems.at[r],
                    recv_sem=s2_recv_sems.at[grp],
                    device_id=(d,),
                    device_id_type=pl.DeviceIdType.MESH,
                )
                wait.wait_send()

    return pl.pallas_call(
        body,
        out_shape=jax.ShapeDtypeStruct((m_out, n), jnp.float32),
        in_specs=[
            pl.BlockSpec(memory_space=pltpu.VMEM),
            pl.BlockSpec(memory_space=pltpu.VMEM),
        ],
        out_specs=pl.BlockSpec(memory_space=pltpu.VMEM),
        scratch_shapes=[
            pltpu.VMEM((G_SZ, N_COLL, m_out, n), jnp.bfloat16),
            pltpu.VMEM((G_SZ, N_COLL, m_out, n), jnp.bfloat16),
            pltpu.VMEM((N_COLL, m_out, n), jnp.bfloat16),
            pltpu.VMEM((N_GRP, m_out, n), jnp.bfloat16),
            pltpu.SemaphoreType.DMA((G_SZ,)),
            pltpu.SemaphoreType.DMA((G_SZ,)),
            pltpu.SemaphoreType.DMA((N_COLL,)),
            pltpu.SemaphoreType.DMA((N_GRP,)),
        ],
        compiler_params=pltpu.CompilerParams(collective_id=0),
    )(A, B)


# device time: 17718 ns/iter; 1.0562x vs baseline; 1.0036x over previous
import jax
import jax.numpy as jnp
from jax import lax
from jax.experimental import pallas as pl
from jax.experimental.pallas import tpu as pltpu

N_DEV = 32
G_SZ = 8
N_GRP = N_DEV // G_SZ
D_PER = G_SZ // N_GRP
N_COLL = N_DEV // G_SZ


def kernel(A, B):
    m, _ = A.shape
    _, n = B.shape
    m_out = m // N_DEV

    def body(
        a_ref, b_ref, out_ref,
        chunks_ref, s1_recv_ref, gchunks_ref, s2_recv_ref,
        s1_send_sems, s1_recv_sems, s2_send_sems, s2_recv_sems,
    ):
        my = lax.axis_index("i")
        slot = lax.rem(my, G_SZ)
        grp = lax.div(my, G_SZ)
        base = my - slot
        coll = lax.div(my, N_COLL)

        barrier_sem = pltpu.get_barrier_semaphore()
        for k in range(1, G_SZ):
            pl.semaphore_signal(
                barrier_sem, inc=1,
                device_id=(base + lax.rem(slot + k, G_SZ),),
                device_id_type=pl.DeviceIdType.MESH,
            )
        for k in range(1, N_GRP):
            pl.semaphore_signal(
                barrier_sem, inc=1,
                device_id=(lax.rem(grp + k, N_GRP) * G_SZ + coll,),
                device_id_type=pl.DeviceIdType.MESH,
            )

        partial = jnp.dot(
            a_ref[...].astype(jnp.bfloat16),
            b_ref[...].astype(jnp.bfloat16),
            preferred_element_type=jnp.float32,
        )
        p_blocks = partial.reshape(G_SZ, N_COLL, m_out, n)
        chunks_ref[...] = p_blocks.astype(jnp.bfloat16)

        targets_local = lax.div(slot, 2) == grp

        @pl.when(targets_local)
        def _():
            pl.semaphore_wait(barrier_sem, G_SZ - 1)

        @pl.when(jnp.logical_not(targets_local))
        def _():
            pl.semaphore_wait(barrier_sem, G_SZ - 1 + N_COLL)

        sends = []
        for k in range(1, G_SZ):
            j = lax.rem(slot + k, G_SZ)
            rdma = pltpu.make_async_remote_copy(
                src_ref=chunks_ref.at[j],
                dst_ref=s1_recv_ref.at[slot],
                send_sem=s1_send_sems.at[k],
                recv_sem=s1_recv_sems.at[slot],
                device_id=(base + j,),
                device_id_type=pl.DeviceIdType.MESH,
            )
            rdma.start()
            sends.append(rdma)

        acc = chunks_ref[pl.ds(slot, 1)][0].astype(jnp.float32)
        for k in range(1, G_SZ):
            j = lax.rem(slot + k, G_SZ)
            recv = pltpu.make_async_remote_copy(
                src_ref=chunks_ref.at[j],
                dst_ref=s1_recv_ref.at[j],
                send_sem=s1_send_sems.at[0],
                recv_sem=s1_recv_sems.at[j],
                device_id=(base + j,),
                device_id_type=pl.DeviceIdType.MESH,
            )
            recv.wait_recv()
            acc = acc + s1_recv_ref[pl.ds(j, 1)][0].astype(jnp.float32)

        gchunks_ref[...] = acc.astype(jnp.bfloat16)

        for r in range(N_COLL):
            d = N_COLL * slot + r

            @pl.when(d != my)
            def _():
                rdma = pltpu.make_async_remote_copy(
                    src_ref=gchunks_ref.at[r],
                    dst_ref=s2_recv_ref.at[grp],
                    send_sem=s2_send_sems.at[r],
                    recv_sem=s2_recv_sems.at[grp],
                    device_id=(d,),
                    device_id_type=pl.DeviceIdType.MESH,
                )
                rdma.start()

            @pl.when(d == my)
            def _():
                s2_recv_ref[pl.ds(grp, 1)] = gchunks_ref[pl.ds(r, 1)]

        for g in range(N_GRP):
            x = g * G_SZ + coll

            @pl.when(x != my)
            def _():
                recv = pltpu.make_async_remote_copy(
                    src_ref=gchunks_ref.at[0],
                    dst_ref=s2_recv_ref.at[g],
                    send_sem=s2_send_sems.at[0],
                    recv_sem=s2_recv_sems.at[g],
                    device_id=(x,),
                    device_id_type=pl.DeviceIdType.MESH,
                )
                recv.wait_recv()

        out_ref[...] = jnp.sum(s2_recv_ref[...].astype(jnp.float32), axis=0)

        for rdma in sends:
            rdma.wait_send()
        for r in range(N_COLL):
            d = N_COLL * slot + r

            @pl.when(d != my)
            def _():
                wait = pltpu.make_async_remote_copy(
                    src_ref=gchunks_ref.at[r],
                    dst_ref=s2_recv_ref.at[grp],
                    send_sem=s2_send_sems.at[r],
                    recv_sem=s2_recv_sems.at[grp],
                    device_id=(d,),
                    device_id_type=pl.DeviceIdType.MESH,
                )
                wait.wait_send()

    return pl.pallas_call(
        body,
        out_shape=jax.ShapeDtypeStruct((m_out, n), jnp.float32),
        in_specs=[
            pl.BlockSpec(memory_space=pltpu.VMEM),
            pl.BlockSpec(memory_space=pltpu.VMEM),
        ],
        out_specs=pl.BlockSpec(memory_space=pltpu.VMEM),
        scratch_shapes=[
            pltpu.VMEM((G_SZ, N_COLL, m_out, n), jnp.bfloat16),
            pltpu.VMEM((G_SZ, N_COLL, m_out, n), jnp.bfloat16),
            pltpu.VMEM((N_COLL, m_out, n), jnp.bfloat16),
            pltpu.VMEM((N_GRP, m_out, n), jnp.bfloat16),
            pltpu.SemaphoreType.DMA((G_SZ,)),
            pltpu.SemaphoreType.DMA((G_SZ,)),
            pltpu.SemaphoreType.DMA((N_COLL,)),
            pltpu.SemaphoreType.DMA((N_GRP,)),
        ],
        compiler_params=pltpu.CompilerParams(collective_id=0),
    )(A, B)
